# baseline (speedup 1.0000x reference)
import functools

import jax
import jax.numpy as jnp
from jax import lax
from jax.experimental import pallas as pl
from jax.experimental.pallas import tpu as pltpu
from jax.experimental.pallas import tpu_sc as plsc

NUM_CLASSES = 1000
D = 256
DH = 128
B = 65536
MOM = 0.9

NC = 2
NS = 16
NW = NC * NS
ROWS_PER_TILE = B // NW
CHUNK = 64
NCHUNK = ROWS_PER_TILE // CHUNK
ACC_ROWS = NUM_CLASSES
STRIPE = 64
LAST_STRIPE = ACC_ROWS - 15 * STRIPE
HROWS = 1008

UNITS = tuple((k // 3, k % 3) for k in range(12))


def _sc_accumulate(zeros, labels1, ff, fr, ft):
  mesh = plsc.VectorSubcoreMesh(core_axis_name="c", subcore_axis_name="s")
  pf32 = jnp.float32

  @functools.partial(
      pl.kernel,
      mesh=mesh,
      out_type=(
          jax.ShapeDtypeStruct((NC, ACC_ROWS, D), pf32),
          jax.ShapeDtypeStruct((NC, ACC_ROWS, D), pf32),
          jax.ShapeDtypeStruct((NC, ACC_ROWS, D), pf32),
          jax.ShapeDtypeStruct((NW * HROWS,), pf32),
      ),
      scratch_types=[
          pltpu.VMEM((CHUNK,), jnp.int32),
          pltpu.VMEM((CHUNK,), jnp.int32),
          pltpu.VMEM((CHUNK, DH), pf32),
          pltpu.VMEM((CHUNK, DH), pf32),
          pltpu.VMEM((CHUNK, DH), pf32),
          pltpu.VMEM((CHUNK, DH), pf32),
          pltpu.VMEM((CHUNK, DH), pf32),
          pltpu.VMEM((CHUNK, DH), pf32),
          pltpu.VMEM((CHUNK, DH), pf32),
          pltpu.VMEM((CHUNK, DH), pf32),
          pltpu.VMEM((HROWS * 16,), pf32),
          pltpu.VMEM((HROWS,), pf32),
          pltpu.VMEM_SHARED((ACC_ROWS, DH), pf32),
          pltpu.VMEM_SHARED((ACC_ROWS, DH), pf32),
          pltpu.VMEM_SHARED((ACC_ROWS, DH), pf32),
          pltpu.VMEM_SHARED((ACC_ROWS, DH), pf32),
          pltpu.VMEM_SHARED((ACC_ROWS, DH), pf32),
          pltpu.VMEM_SHARED((ACC_ROWS, DH), pf32),
          pltpu.SemaphoreType.DMA,
          pltpu.SemaphoreType.DMA,
      ],
  )
  def body(zeros_hbm, labels_hbm, ff_hbm, fr_hbm, ft_hbm,
           out_f, out_r, out_t, out_c,
           lab0, lab1, lo0, hi0, lo1, hi1, lo2, hi2, lo3, hi3, hist, cvec,
           acc_f0, acc_f1, acc_r0, acc_r1, acc_t0, acc_t1,
           gsem, ssem):
    cid = lax.axis_index("c")
    sid = lax.axis_index("s")
    wid = cid * NS + sid
    r0 = sid * STRIPE
    base = wid * ROWS_PER_TILE

    labs = (lab0, lab1)
    bl = (lo0, lo1, lo2, lo3)
    bh = (hi0, hi1, hi2, hi3)
    streams = ((ff_hbm, acc_f0, acc_f1),
               (fr_hbm, acc_r0, acc_r1),
               (ft_hbm, acc_t0, acc_t1))

    def zero_all(rows):
      zsrc = zeros_hbm.at[rows, pl.ds(0, DH)]
      pltpu.sync_copy(zsrc, acc_f0.at[rows])
      pltpu.sync_copy(zsrc, acc_f1.at[rows])
      pltpu.sync_copy(zsrc, acc_r0.at[rows])
      pltpu.sync_copy(zsrc, acc_r1.at[rows])
      pltpu.sync_copy(zsrc, acc_t0.at[rows])
      pltpu.sync_copy(zsrc, acc_t1.at[rows])

    @pl.when(sid < NS - 1)
    def _():
      zero_all(pl.ds(r0, STRIPE))

    @pl.when(sid == NS - 1)
    def _():
      zero_all(pl.ds((NS - 1) * STRIPE, LAST_STRIPE))

    one16 = jnp.ones((16,), pf32)
    zero16 = jnp.zeros((16,), pf32)

    def zero_hist(i, carry):
      hist[pl.ds(i * 16, 16)] = zero16
      return carry
    lax.fori_loop(0, HROWS, zero_hist, 0)

    plsc.subcore_barrier()

    def rows_of(j):
      return pl.ds(base + j * CHUNK, CHUNK)

    def issue_gather(j, s, p, dj):
      src = streams[s][0]
      if s == 0:
        pltpu.async_copy(labels_hbm.at[rows_of(j)], labs[dj], gsem)
      pltpu.async_copy(src.at[rows_of(j), pl.ds(0, DH)], bl[p], gsem)
      pltpu.async_copy(src.at[rows_of(j), pl.ds(DH, DH)], bh[p], gsem)

    def wait_gather(s, p, dj):
      src = streams[s][0]
      if s == 0:
        pltpu.make_async_copy(labels_hbm.at[pl.ds(0, CHUNK)], labs[dj], gsem).wait()
      pltpu.make_async_copy(src.at[pl.ds(0, CHUNK), pl.ds(0, DH)], bl[p], gsem).wait()
      pltpu.make_async_copy(src.at[pl.ds(0, CHUNK), pl.ds(DH, DH)], bh[p], gsem).wait()

    def issue_scatter(s, p, dj):
      _, a0, a1 = streams[s]
      pltpu.async_copy(bl[p], a0.at[labs[dj]], ssem, add=True)
      pltpu.async_copy(bh[p], a1.at[labs[dj]], ssem, add=True)

    def wait_scatter(s, p, dj):
      _, a0, a1 = streams[s]
      pltpu.make_async_copy(bl[p], a0.at[labs[dj]], ssem).wait()
      pltpu.make_async_copy(bh[p], a1.at[labs[dj]], ssem).wait()

    def hist_groups(dj, groups):
      for g in groups:
        row = labs[dj][pl.ds(g * 16, 16)]
        for l in range(16):
          xo = row[l] * 16
          hist[pl.ds(xo, 16)] = hist[pl.ds(xo, 16)] + one16

    def step(k, j0, first_quad):
      dj, s = UNITS[k]
      p = k % 4
      wait_gather(s, p, dj % 2)
      issue_scatter(s, p, dj % 2)
      if k >= 2:
        pdj, ps = UNITS[k - 2]
        wait_scatter(ps, (k - 2) % 4, pdj % 2)
      elif not first_quad:
        pdj, ps = UNITS[k + 10]
        wait_scatter(ps, (k + 10) % 4, pdj % 2)
      if k < 10:
        ndj, ns = UNITS[k + 2]
        issue_gather(j0 + ndj, ns, (k + 2) % 4, ndj % 2)
      else:
        nk = k - 10
        ndj, ns = UNITS[nk]

        @pl.when(j0 + 4 < NCHUNK)
        def _():
          issue_gather(j0 + 4 + ndj, ns, nk % 4, ndj % 2)
      if s == 1:
        hist_groups(dj % 2, (0, 1))
      elif s == 2:
        hist_groups(dj % 2, (2, 3))

    issue_gather(0, 0, 0, 0)
    issue_gather(0, 1, 1, 0)
    for k in range(12):
      step(k, 0, first_quad=True)

    @pl.loop(4, NCHUNK, step=4)
    def _quad(j0):
      for k in range(12):
        step(k, j0, first_quad=False)

    wait_scatter(UNITS[10][1], 10 % 4, UNITS[10][0] % 2)
    wait_scatter(UNITS[11][1], 11 % 4, UNITS[11][0] % 2)

    plsc.subcore_barrier()

    def write_all(rows):
      pltpu.sync_copy(acc_f0.at[rows], out_f.at[cid, rows, pl.ds(0, DH)])
      pltpu.sync_copy(acc_f1.at[rows], out_f.at[cid, rows, pl.ds(DH, DH)])
      pltpu.sync_copy(acc_r0.at[rows], out_r.at[cid, rows, pl.ds(0, DH)])
      pltpu.sync_copy(acc_r1.at[rows], out_r.at[cid, rows, pl.ds(DH, DH)])
      pltpu.sync_copy(acc_t0.at[rows], out_t.at[cid, rows, pl.ds(0, DH)])
      pltpu.sync_copy(acc_t1.at[rows], out_t.at[cid, rows, pl.ds(DH, DH)])

    @pl.when(sid < NS - 1)
    def _():
      write_all(pl.ds(r0, STRIPE))

    @pl.when(sid == NS - 1)
    def _():
      write_all(pl.ds((NS - 1) * STRIPE, LAST_STRIPE))

    iota16 = lax.iota(jnp.int32, 16)

    def extract(g, carry):
      acc = zero16
      for l in range(16):
        acc = jnp.where(iota16 == l, hist[pl.ds((g * 16 + l) * 16, 16)], acc)
      cvec[pl.ds(g * 16, 16)] = acc
      return carry
    lax.fori_loop(0, HROWS // 16, extract, 0)
    pltpu.sync_copy(cvec, out_c.at[pl.ds(wid * HROWS, HROWS)])

  return body(zeros, labels1, ff, fr, ft)


def _tc_combine_body(pf, pr, pt, pc, prf, prr, prt, o_ref):
  cnt = jnp.sum(pc[...], axis=0)
  denom = jnp.maximum(cnt, 1.0)
  w_new = 1.0 - MOM
  o_ref[0] = w_new * ((pf[0] + pf[1]) / denom) + MOM * prf[...]
  o_ref[1] = w_new * ((pr[0] + pr[1]) / denom) + MOM * prr[...]
  o_ref[2] = w_new * ((pt[0] + pt[1]) / denom) + MOM * prt[...]


def kernel(feat_fusion, feat_rgb, feat_t, labels, proto_fusion, proto_rgb,
           proto_t):
  labels1 = labels.astype(jnp.int32)
  zeros = jnp.zeros((STRIPE * NS, DH), jnp.float32)

  pf, pr, pt, pc = _sc_accumulate(zeros, labels1, feat_fusion, feat_rgb,
                                  feat_t)
  pc2 = pc.reshape(NW, HROWS)[:, :NUM_CLASSES, None]

  out = pl.pallas_call(
      _tc_combine_body,
      out_shape=jax.ShapeDtypeStruct((3, ACC_ROWS, D), jnp.float32),
  )(pf, pr, pt, pc2, proto_fusion, proto_rgb, proto_t)
  return out


# speedup vs baseline: 1.0906x; 1.0906x over previous
import functools

import jax
import jax.numpy as jnp
from jax import lax
from jax.experimental import pallas as pl
from jax.experimental.pallas import tpu as pltpu
from jax.experimental.pallas import tpu_sc as plsc

NUM_CLASSES = 1000
D = 256
DH = 128
B = 65536
MOM = 0.9

NC = 2
NS = 16
NW = NC * NS
ROWS_PER_TILE = B // NW
CHUNK = 64
NCHUNK = ROWS_PER_TILE // CHUNK
ACC_ROWS = NUM_CLASSES
STRIPE = 64
LAST_STRIPE = ACC_ROWS - 15 * STRIPE
CNT_W = 128

UNITS = tuple((k // 3, k % 3) for k in range(12))


def _sc_accumulate(zeros, labels1, ff, fr, ft):
  mesh = plsc.VectorSubcoreMesh(core_axis_name="c", subcore_axis_name="s")
  pf32 = jnp.float32

  @functools.partial(
      pl.kernel,
      mesh=mesh,
      out_type=(
          jax.ShapeDtypeStruct((NC, ACC_ROWS, D), pf32),
          jax.ShapeDtypeStruct((NC, ACC_ROWS, D), pf32),
          jax.ShapeDtypeStruct((NC, ACC_ROWS, D), pf32),
          jax.ShapeDtypeStruct((NC, ACC_ROWS, CNT_W), pf32),
      ),
      scratch_types=[
          pltpu.VMEM((CHUNK,), jnp.int32),
          pltpu.VMEM((CHUNK,), jnp.int32),
          pltpu.VMEM((CHUNK, DH), pf32),
          pltpu.VMEM((CHUNK, DH), pf32),
          pltpu.VMEM((CHUNK, DH), pf32),
          pltpu.VMEM((CHUNK, DH), pf32),
          pltpu.VMEM((CHUNK, DH), pf32),
          pltpu.VMEM((CHUNK, DH), pf32),
          pltpu.VMEM((CHUNK, DH), pf32),
          pltpu.VMEM((CHUNK, DH), pf32),
          pltpu.VMEM((CHUNK, CNT_W), pf32),
          pltpu.VMEM_SHARED((ACC_ROWS, DH), pf32),
          pltpu.VMEM_SHARED((ACC_ROWS, DH), pf32),
          pltpu.VMEM_SHARED((ACC_ROWS, DH), pf32),
          pltpu.VMEM_SHARED((ACC_ROWS, DH), pf32),
          pltpu.VMEM_SHARED((ACC_ROWS, DH), pf32),
          pltpu.VMEM_SHARED((ACC_ROWS, DH), pf32),
          pltpu.VMEM_SHARED((ACC_ROWS, CNT_W), pf32),
          pltpu.SemaphoreType.DMA,
          pltpu.SemaphoreType.DMA,
      ],
  )
  def body(zeros_hbm, labels_hbm, ff_hbm, fr_hbm, ft_hbm,
           out_f, out_r, out_t, out_c,
           lab0, lab1, lo0, hi0, lo1, hi1, lo2, hi2, lo3, hi3, ones_v,
           acc_f0, acc_f1, acc_r0, acc_r1, acc_t0, acc_t1, acc_c,
           gsem, ssem):
    cid = lax.axis_index("c")
    sid = lax.axis_index("s")
    wid = cid * NS + sid
    r0 = sid * STRIPE
    base = wid * ROWS_PER_TILE

    labs = (lab0, lab1)
    bl = (lo0, lo1, lo2, lo3)
    bh = (hi0, hi1, hi2, hi3)
    streams = ((ff_hbm, acc_f0, acc_f1),
               (fr_hbm, acc_r0, acc_r1),
               (ft_hbm, acc_t0, acc_t1))

    def zero_all(rows):
      zsrc = zeros_hbm.at[rows, pl.ds(0, DH)]
      pltpu.sync_copy(zsrc, acc_f0.at[rows])
      pltpu.sync_copy(zsrc, acc_f1.at[rows])
      pltpu.sync_copy(zsrc, acc_r0.at[rows])
      pltpu.sync_copy(zsrc, acc_r1.at[rows])
      pltpu.sync_copy(zsrc, acc_t0.at[rows])
      pltpu.sync_copy(zsrc, acc_t1.at[rows])
      pltpu.sync_copy(zsrc, acc_c.at[rows])

    @pl.when(sid < NS - 1)
    def _():
      zero_all(pl.ds(r0, STRIPE))

    @pl.when(sid == NS - 1)
    def _():
      zero_all(pl.ds((NS - 1) * STRIPE, LAST_STRIPE))

    one16 = jnp.ones((16,), pf32)

    def fill_ones(q, carry):
      ones_v[q // (CNT_W // 16), pl.ds((q % (CNT_W // 16)) * 16, 16)] = one16
      return carry
    lax.fori_loop(0, CHUNK * (CNT_W // 16), fill_ones, 0)

    plsc.subcore_barrier()

    def rows_of(j):
      return pl.ds(base + j * CHUNK, CHUNK)

    def issue_gather(j, s, p, dj):
      src = streams[s][0]
      if s == 0:
        pltpu.async_copy(labels_hbm.at[rows_of(j)], labs[dj], gsem)
      pltpu.async_copy(src.at[rows_of(j), pl.ds(0, DH)], bl[p], gsem)
      pltpu.async_copy(src.at[rows_of(j), pl.ds(DH, DH)], bh[p], gsem)

    def wait_gather(s, p, dj):
      src = streams[s][0]
      if s == 0:
        pltpu.make_async_copy(labels_hbm.at[pl.ds(0, CHUNK)], labs[dj], gsem).wait()
      pltpu.make_async_copy(src.at[pl.ds(0, CHUNK), pl.ds(0, DH)], bl[p], gsem).wait()
      pltpu.make_async_copy(src.at[pl.ds(0, CHUNK), pl.ds(DH, DH)], bh[p], gsem).wait()

    def issue_scatter(s, p, dj):
      _, a0, a1 = streams[s]
      pltpu.async_copy(bl[p], a0.at[labs[dj]], ssem, add=True)
      pltpu.async_copy(bh[p], a1.at[labs[dj]], ssem, add=True)
      if s == 2:
        pltpu.async_copy(ones_v, acc_c.at[labs[dj]], ssem, add=True)

    def wait_scatter(s, p, dj):
      _, a0, a1 = streams[s]
      pltpu.make_async_copy(bl[p], a0.at[labs[dj]], ssem).wait()
      pltpu.make_async_copy(bh[p], a1.at[labs[dj]], ssem).wait()
      if s == 2:
        pltpu.make_async_copy(ones_v, acc_c.at[labs[dj]], ssem).wait()

    def step(k, j0, first_quad):
      dj, s = UNITS[k]
      p = k % 4
      wait_gather(s, p, dj % 2)
      if k >= 2:
        pdj, ps = UNITS[k - 2]
        wait_scatter(ps, (k - 2) % 4, pdj % 2)
      elif not first_quad:
        pdj, ps = UNITS[k + 10]
        wait_scatter(ps, (k + 10) % 4, pdj % 2)
      if k < 10:
        ndj, ns = UNITS[k + 2]
        issue_gather(j0 + ndj, ns, (k + 2) % 4, ndj % 2)
      else:
        nk = k - 10
        ndj, ns = UNITS[nk]

        @pl.when(j0 + 4 < NCHUNK)
        def _():
          issue_gather(j0 + 4 + ndj, ns, nk % 4, ndj % 2)
      issue_scatter(s, p, dj % 2)

    issue_gather(0, 0, 0, 0)
    issue_gather(0, 1, 1, 0)
    for k in range(12):
      step(k, 0, first_quad=True)

    @pl.loop(4, NCHUNK, step=4)
    def _quad(j0):
      for k in range(12):
        step(k, j0, first_quad=False)

    wait_scatter(UNITS[10][1], 10 % 4, UNITS[10][0] % 2)
    wait_scatter(UNITS[11][1], 11 % 4, UNITS[11][0] % 2)

    plsc.subcore_barrier()

    def write_all(rows):
      pltpu.sync_copy(acc_f0.at[rows], out_f.at[cid, rows, pl.ds(0, DH)])
      pltpu.sync_copy(acc_f1.at[rows], out_f.at[cid, rows, pl.ds(DH, DH)])
      pltpu.sync_copy(acc_r0.at[rows], out_r.at[cid, rows, pl.ds(0, DH)])
      pltpu.sync_copy(acc_r1.at[rows], out_r.at[cid, rows, pl.ds(DH, DH)])
      pltpu.sync_copy(acc_t0.at[rows], out_t.at[cid, rows, pl.ds(0, DH)])
      pltpu.sync_copy(acc_t1.at[rows], out_t.at[cid, rows, pl.ds(DH, DH)])
      pltpu.sync_copy(acc_c.at[rows], out_c.at[cid, rows])

    @pl.when(sid < NS - 1)
    def _():
      write_all(pl.ds(r0, STRIPE))

    @pl.when(sid == NS - 1)
    def _():
      write_all(pl.ds((NS - 1) * STRIPE, LAST_STRIPE))

  return body(zeros, labels1, ff, fr, ft)


def _tc_combine_body(pf, pr, pt, pc, prf, prr, prt, o_ref):
  cnt = pc[0] + pc[1]
  denom = jnp.maximum(cnt[:, 0:1], 1.0)
  w_new = 1.0 - MOM
  o_ref[0] = w_new * ((pf[0] + pf[1]) / denom) + MOM * prf[...]
  o_ref[1] = w_new * ((pr[0] + pr[1]) / denom) + MOM * prr[...]
  o_ref[2] = w_new * ((pt[0] + pt[1]) / denom) + MOM * prt[...]


def kernel(feat_fusion, feat_rgb, feat_t, labels, proto_fusion, proto_rgb,
           proto_t):
  labels1 = labels.astype(jnp.int32)
  zeros = jnp.zeros((STRIPE * NS, DH), jnp.float32)

  pf, pr, pt, pc = _sc_accumulate(zeros, labels1, feat_fusion, feat_rgb,
                                  feat_t)

  out = pl.pallas_call(
      _tc_combine_body,
      out_shape=jax.ShapeDtypeStruct((3, ACC_ROWS, D), jnp.float32),
  )(pf, pr, pt, pc, proto_fusion, proto_rgb, proto_t)
  return out


# speedup vs baseline: 1.0917x; 1.0011x over previous
import functools

import jax
import jax.numpy as jnp
from jax import lax
from jax.experimental import pallas as pl
from jax.experimental.pallas import tpu as pltpu
from jax.experimental.pallas import tpu_sc as plsc

NUM_CLASSES = 1000
D = 256
DH = 128
B = 65536
MOM = 0.9

NC = 2
NS = 16
NW = NC * NS
ROWS_PER_TILE = B // NW
CHUNK = 64
NCHUNK = ROWS_PER_TILE // CHUNK
ACC_ROWS = NUM_CLASSES
STRIPE = 64
LAST_STRIPE = ACC_ROWS - 15 * STRIPE
CNT_W = 128

UNITS = tuple((k // 3, k % 3) for k in range(12))


def _sc_accumulate(zeros, labels1, ff, fr, ft):
  mesh = plsc.VectorSubcoreMesh(core_axis_name="c", subcore_axis_name="s")
  pf32 = jnp.float32

  @functools.partial(
      pl.kernel,
      mesh=mesh,
      out_type=(
          jax.ShapeDtypeStruct((NC, ACC_ROWS, D), pf32),
          jax.ShapeDtypeStruct((NC, ACC_ROWS, D), pf32),
          jax.ShapeDtypeStruct((NC, ACC_ROWS, D), pf32),
          jax.ShapeDtypeStruct((NC, ACC_ROWS, CNT_W), pf32),
      ),
      scratch_types=[
          pltpu.VMEM((CHUNK,), jnp.int32),
          pltpu.VMEM((CHUNK,), jnp.int32),
          pltpu.VMEM((CHUNK, DH), pf32),
          pltpu.VMEM((CHUNK, DH), pf32),
          pltpu.VMEM((CHUNK, DH), pf32),
          pltpu.VMEM((CHUNK, DH), pf32),
          pltpu.VMEM((CHUNK, DH), pf32),
          pltpu.VMEM((CHUNK, DH), pf32),
          pltpu.VMEM((CHUNK, DH), pf32),
          pltpu.VMEM((CHUNK, DH), pf32),
          pltpu.VMEM((CHUNK, CNT_W), pf32),
          pltpu.VMEM_SHARED((ACC_ROWS, DH), pf32),
          pltpu.VMEM_SHARED((ACC_ROWS, DH), pf32),
          pltpu.VMEM_SHARED((ACC_ROWS, DH), pf32),
          pltpu.VMEM_SHARED((ACC_ROWS, DH), pf32),
          pltpu.VMEM_SHARED((ACC_ROWS, DH), pf32),
          pltpu.VMEM_SHARED((ACC_ROWS, DH), pf32),
          pltpu.VMEM_SHARED((ACC_ROWS, CNT_W), pf32),
          pltpu.SemaphoreType.DMA,
          pltpu.SemaphoreType.DMA,
      ],
  )
  def body(zeros_hbm, labels_hbm, ff_hbm, fr_hbm, ft_hbm,
           out_f, out_r, out_t, out_c,
           lab0, lab1, lo0, hi0, lo1, hi1, lo2, hi2, lo3, hi3, ones_v,
           acc_f0, acc_f1, acc_r0, acc_r1, acc_t0, acc_t1, acc_c,
           gsem, ssem):
    cid = lax.axis_index("c")
    sid = lax.axis_index("s")
    wid = cid * NS + sid
    r0 = sid * STRIPE
    base = wid * ROWS_PER_TILE

    labs = (lab0, lab1)
    bl = (lo0, lo1, lo2, lo3)
    bh = (hi0, hi1, hi2, hi3)
    streams = ((ff_hbm, acc_f0, acc_f1),
               (fr_hbm, acc_r0, acc_r1),
               (ft_hbm, acc_t0, acc_t1))

    def zero_all(rows):
      zsrc = zeros_hbm.at[rows, pl.ds(0, DH)]
      pltpu.sync_copy(zsrc, acc_f0.at[rows])
      pltpu.sync_copy(zsrc, acc_f1.at[rows])
      pltpu.sync_copy(zsrc, acc_r0.at[rows])
      pltpu.sync_copy(zsrc, acc_r1.at[rows])
      pltpu.sync_copy(zsrc, acc_t0.at[rows])
      pltpu.sync_copy(zsrc, acc_t1.at[rows])
      pltpu.sync_copy(zsrc, acc_c.at[rows])

    @pl.when(sid < NS - 1)
    def _():
      zero_all(pl.ds(r0, STRIPE))

    @pl.when(sid == NS - 1)
    def _():
      zero_all(pl.ds((NS - 1) * STRIPE, LAST_STRIPE))

    one16 = jnp.ones((16,), pf32)

    def fill_ones(q, carry):
      ones_v[q // (CNT_W // 16), pl.ds((q % (CNT_W // 16)) * 16, 16)] = one16
      return carry
    lax.fori_loop(0, CHUNK * (CNT_W // 16), fill_ones, 0)

    plsc.subcore_barrier()

    def rows_of(j):
      return pl.ds(base + j * CHUNK, CHUNK)

    def issue_gather(j, s, p, dj):
      src = streams[s][0]
      if s == 0:
        pltpu.async_copy(labels_hbm.at[rows_of(j)], labs[dj], gsem)
      pltpu.async_copy(src.at[rows_of(j), pl.ds(0, DH)], bl[p], gsem)
      pltpu.async_copy(src.at[rows_of(j), pl.ds(DH, DH)], bh[p], gsem)

    def wait_gather(s, p, dj):
      src = streams[s][0]
      if s == 0:
        pltpu.make_async_copy(labels_hbm.at[pl.ds(0, CHUNK)], labs[dj], gsem).wait()
      pltpu.make_async_copy(src.at[pl.ds(0, CHUNK), pl.ds(0, DH)], bl[p], gsem).wait()
      pltpu.make_async_copy(src.at[pl.ds(0, CHUNK), pl.ds(DH, DH)], bh[p], gsem).wait()

    def issue_scatter(s, p, dj):
      _, a0, a1 = streams[s]
      pltpu.async_copy(bl[p], a0.at[labs[dj]], ssem, add=True)
      pltpu.async_copy(bh[p], a1.at[labs[dj]], ssem, add=True)
      if s == 2:
        pltpu.async_copy(ones_v, acc_c.at[labs[dj]], ssem, add=True)

    def wait_scatter(s, p, dj):
      _, a0, a1 = streams[s]
      pltpu.make_async_copy(bl[p], a0.at[labs[dj]], ssem).wait()
      pltpu.make_async_copy(bh[p], a1.at[labs[dj]], ssem).wait()
      if s == 2:
        pltpu.make_async_copy(ones_v, acc_c.at[labs[dj]], ssem).wait()

    def step(k, j0, first_quad):
      dj, s = UNITS[k]
      p = k % 4
      wait_gather(s, p, dj % 2)
      issue_scatter(s, p, dj % 2)
      if k >= 2:
        pdj, ps = UNITS[k - 2]
        wait_scatter(ps, (k - 2) % 4, pdj % 2)
      elif not first_quad:
        pdj, ps = UNITS[k + 10]
        wait_scatter(ps, (k + 10) % 4, pdj % 2)
      if k < 10:
        ndj, ns = UNITS[k + 2]
        issue_gather(j0 + ndj, ns, (k + 2) % 4, ndj % 2)
      else:
        nk = k - 10
        ndj, ns = UNITS[nk]

        @pl.when(j0 + 4 < NCHUNK)
        def _():
          issue_gather(j0 + 4 + ndj, ns, nk % 4, ndj % 2)

    issue_gather(0, 0, 0, 0)
    issue_gather(0, 1, 1, 0)
    for k in range(12):
      step(k, 0, first_quad=True)

    @pl.loop(4, NCHUNK, step=4)
    def _quad(j0):
      for k in range(12):
        step(k, j0, first_quad=False)

    wait_scatter(UNITS[10][1], 10 % 4, UNITS[10][0] % 2)
    wait_scatter(UNITS[11][1], 11 % 4, UNITS[11][0] % 2)

    plsc.subcore_barrier()

    def write_all(rows):
      pltpu.sync_copy(acc_f0.at[rows], out_f.at[cid, rows, pl.ds(0, DH)])
      pltpu.sync_copy(acc_f1.at[rows], out_f.at[cid, rows, pl.ds(DH, DH)])
      pltpu.sync_copy(acc_r0.at[rows], out_r.at[cid, rows, pl.ds(0, DH)])
      pltpu.sync_copy(acc_r1.at[rows], out_r.at[cid, rows, pl.ds(DH, DH)])
      pltpu.sync_copy(acc_t0.at[rows], out_t.at[cid, rows, pl.ds(0, DH)])
      pltpu.sync_copy(acc_t1.at[rows], out_t.at[cid, rows, pl.ds(DH, DH)])
      pltpu.sync_copy(acc_c.at[rows], out_c.at[cid, rows])

    @pl.when(sid < NS - 1)
    def _():
      write_all(pl.ds(r0, STRIPE))

    @pl.when(sid == NS - 1)
    def _():
      write_all(pl.ds((NS - 1) * STRIPE, LAST_STRIPE))

  return body(zeros, labels1, ff, fr, ft)


def _tc_combine_body(pf, pr, pt, pc, prf, prr, prt, o_ref):
  cnt = pc[0] + pc[1]
  denom = jnp.maximum(cnt[:, 0:1], 1.0)
  w_new = 1.0 - MOM
  o_ref[0] = w_new * ((pf[0] + pf[1]) / denom) + MOM * prf[...]
  o_ref[1] = w_new * ((pr[0] + pr[1]) / denom) + MOM * prr[...]
  o_ref[2] = w_new * ((pt[0] + pt[1]) / denom) + MOM * prt[...]


def kernel(feat_fusion, feat_rgb, feat_t, labels, proto_fusion, proto_rgb,
           proto_t):
  labels1 = labels.astype(jnp.int32)
  zeros = jnp.zeros((STRIPE * NS, DH), jnp.float32)

  pf, pr, pt, pc = _sc_accumulate(zeros, labels1, feat_fusion, feat_rgb,
                                  feat_t)

  out = pl.pallas_call(
      _tc_combine_body,
      out_shape=jax.ShapeDtypeStruct((3, ACC_ROWS, D), jnp.float32),
  )(pf, pr, pt, pc, proto_fusion, proto_rgb, proto_t)
  return out
